# baseline (speedup 1.0000x reference)
import functools

import jax
import jax.numpy as jnp
from jax.experimental import pallas as pl
from jax.experimental.pallas import tpu as pltpu
from jax.experimental.pallas import tpu_sc as plsc

H = 1024
FFN = 4096
E = 8
K = 2
T = 4096

BT = 1024
BROW = 384
BF = 2048
G = -(-(T * K + E * (BROW - 1)) // BROW)
NPAD = G * BROW
NF = FFN // BF
CH = 128
NCH = T // CH
SCW = 32


def _dotT(a, b):
    return jax.lax.dot_general(a, b, (((1,), (1,)), ((), ())),
                               preferred_element_type=jnp.float32)


def _router_body(x_ref, win_ref, wgate_ref, h_ref, logits_ref):
    h = _dotT(x_ref[...], win_ref[...])
    h_ref[...] = h
    logits_ref[...] = _dotT(h, wgate_ref[...])


def _index_body(logits_ref, pos0_ref, pos1_ref, w0_ref, w1_ref, te_ref,
                used_ref):
    logits = logits_ref[...]
    probs = jax.nn.softmax(logits, axis=-1)
    lane = jax.lax.broadcasted_iota(jnp.int32, probs.shape, 1)
    p1 = jnp.max(probs, axis=-1, keepdims=True)
    i1 = jnp.argmax(probs, axis=-1)
    oh1 = (lane == i1[:, None]).astype(jnp.float32)
    masked = jnp.where(oh1 > 0, -1.0, probs)
    p2 = jnp.max(masked, axis=-1, keepdims=True)
    i2 = jnp.argmax(masked, axis=-1)
    oh2 = (lane == i2[:, None]).astype(jnp.float32)
    e2 = jnp.exp(p2 - p1)
    w1v = 1.0 / (1.0 + e2)
    w2v = e2 / (1.0 + e2)
    w0_ref[...] = jnp.broadcast_to(w1v, (T, E))
    w1_ref[...] = jnp.broadcast_to(w2v, (T, E))

    oh = oh1 + oh2
    sub = jax.lax.broadcasted_iota(jnp.int32, (CH, CH), 0)
    lan2 = jax.lax.broadcasted_iota(jnp.int32, (CH, CH), 1)
    tri_incl = (lan2 <= sub).astype(jnp.float32)
    incl_chunks = []
    totals = []
    for c in range(NCH):
        blk = oh[c * CH:(c + 1) * CH, :]
        incl = jnp.dot(tri_incl, blk, preferred_element_type=jnp.float32)
        incl_chunks.append(incl)
        totals.append(incl[CH - 1:CH, :])
    tot = jnp.concatenate(totals, axis=0)
    sub3 = jax.lax.broadcasted_iota(jnp.int32, (NCH, NCH), 0)
    lan3 = jax.lax.broadcasted_iota(jnp.int32, (NCH, NCH), 1)
    tri_strict = (lan3 < sub3).astype(jnp.float32)
    excl_tot = jnp.dot(tri_strict, tot, preferred_element_type=jnp.float32)
    incl_all = jnp.concatenate(
        [incl_chunks[c] + excl_tot[c:c + 1, :] for c in range(NCH)], axis=0)
    counts = jnp.sum(tot, axis=0, keepdims=True)
    padded = jnp.floor((counts + (BROW - 1)) / BROW) * BROW
    sub4 = jax.lax.broadcasted_iota(jnp.int32, (E, E), 0)
    lan4 = jax.lax.broadcasted_iota(jnp.int32, (E, E), 1)
    lt = (sub4 < lan4).astype(jnp.float32)
    offsets = jnp.dot(padded, lt, preferred_element_type=jnp.float32)
    excl = incl_all - oh
    pos_e = offsets + excl
    pos0 = jnp.sum(oh1 * pos_e, axis=-1, keepdims=True)
    pos1 = jnp.sum(oh2 * pos_e, axis=-1, keepdims=True)
    pos0_ref[...] = jnp.broadcast_to(pos0, (T, E)).astype(jnp.int32)
    pos1_ref[...] = jnp.broadcast_to(pos1, (T, E)).astype(jnp.int32)
    total_used = jnp.sum(padded, axis=-1, keepdims=True)
    used_ref[...] = jnp.broadcast_to(
        total_used / BROW, (1, E)).astype(jnp.int32)
    g_iota = (jax.lax.broadcasted_iota(jnp.int32, (1, G), 1) * BROW
              ).astype(jnp.float32)
    te = jnp.zeros((1, G), jnp.float32)
    for e in range(E):
        te = te + (g_iota >= offsets[0, e]).astype(jnp.float32)
    te_ref[...] = (te - 1.0).astype(jnp.int32)


NSHARD = 32
NTOK = T // NSHARD
NSUB = NTOK // SCW


def _dispatch(h, pos0, pos1):
    mesh = plsc.VectorSubcoreMesh(core_axis_name="c", subcore_axis_name="s")

    @functools.partial(
        pl.kernel,
        out_type=jax.ShapeDtypeStruct((NPAD, H), jnp.float32),
        mesh=mesh,
        scratch_types=[pltpu.VMEM((1, NTOK), jnp.int32),
                       pltpu.VMEM((1, NTOK), jnp.int32),
                       pltpu.VMEM((SCW, H), jnp.float32),
                       pltpu.SemaphoreType.DMA,
                       pltpu.SemaphoreType.DMA])
    def run(h_hbm, p0_hbm, p1_hbm, hg_hbm, i0b, i1b, rowb, sem0, sem1):
        c = jax.lax.axis_index("c")
        s = jax.lax.axis_index("s")
        base = (c * 16 + s) * NTOK
        pltpu.async_copy(p0_hbm.at[:, pl.ds(base, NTOK)], i0b, sem0).wait()
        pltpu.async_copy(p1_hbm.at[:, pl.ds(base, NTOK)], i1b, sem1).wait()

        @pl.loop(0, NSUB)
        def _(sub):
            r0 = base + sub * SCW
            pltpu.async_copy(h_hbm.at[pl.ds(r0, SCW), :], rowb, sem0).wait()
            cp0 = pltpu.async_copy(
                rowb, hg_hbm.at[i0b.at[0, pl.ds(sub * SCW, SCW)]], sem0)
            cp1 = pltpu.async_copy(
                rowb, hg_hbm.at[i1b.at[0, pl.ds(sub * SCW, SCW)]], sem1)
            cp0.wait()
            cp1.wait()

    return run(h, pos0, pos1)


def _combine_gather(og_a, og_b, pos0, pos1):
    mesh = plsc.VectorSubcoreMesh(core_axis_name="c", subcore_axis_name="s")
    W4 = SCW // 2

    @functools.partial(
        pl.kernel,
        out_type=[jax.ShapeDtypeStruct((T, H), jnp.float32)] * 4,
        mesh=mesh,
        scratch_types=[pltpu.VMEM((1, NTOK), jnp.int32),
                       pltpu.VMEM((1, NTOK), jnp.int32),
                       pltpu.VMEM((W4, H), jnp.float32),
                       pltpu.VMEM((W4, H), jnp.float32),
                       pltpu.VMEM((W4, H), jnp.float32),
                       pltpu.VMEM((W4, H), jnp.float32),
                       pltpu.SemaphoreType.DMA,
                       pltpu.SemaphoreType.DMA,
                       pltpu.SemaphoreType.DMA,
                       pltpu.SemaphoreType.DMA])
    def run(oga_hbm, ogb_hbm, p0_hbm, p1_hbm, o0a_hbm, o1a_hbm, o0b_hbm,
            o1b_hbm, i0b, i1b, rb0a, rb1a, rb0b, rb1b,
            sem0, sem1, sem2, sem3):
        c = jax.lax.axis_index("c")
        s = jax.lax.axis_index("s")
        base = (c * 16 + s) * NTOK
        pltpu.async_copy(p0_hbm.at[:, pl.ds(base, NTOK)], i0b, sem0).wait()
        pltpu.async_copy(p1_hbm.at[:, pl.ds(base, NTOK)], i1b, sem1).wait()

        @pl.loop(0, NTOK // W4)
        def _(sub):
            r0 = base + sub * W4
            i0 = i0b.at[0, pl.ds(sub * W4, W4)]
            i1 = i1b.at[0, pl.ds(sub * W4, W4)]
            cps = [pltpu.async_copy(oga_hbm.at[i0], rb0a, sem0),
                   pltpu.async_copy(oga_hbm.at[i1], rb1a, sem1),
                   pltpu.async_copy(ogb_hbm.at[i0], rb0b, sem2),
                   pltpu.async_copy(ogb_hbm.at[i1], rb1b, sem3)]
            for cp in cps:
                cp.wait()
            cps = [pltpu.async_copy(rb0a, o0a_hbm.at[pl.ds(r0, W4), :], sem0),
                   pltpu.async_copy(rb1a, o1a_hbm.at[pl.ds(r0, W4), :], sem1),
                   pltpu.async_copy(rb0b, o0b_hbm.at[pl.ds(r0, W4), :], sem2),
                   pltpu.async_copy(rb1b, o1b_hbm.at[pl.ds(r0, W4), :], sem3)]
            for cp in cps:
                cp.wait()

    return run(og_a, og_b, pos0, pos1)


def _grouped_body_first(te_ref, used_ref, hg_ref, gate_ref, up_ref, down_ref,
                        og_ref):
    g = pl.program_id(0)

    @pl.when(g < used_ref[0])
    def _():
        hg = hg_ref[...]
        gv = _dotT(hg, gate_ref[0])
        uv = _dotT(hg, up_ref[0])
        p = jax.nn.silu(gv) * uv
        og_ref[...] = jax.lax.dot_general(
            p, down_ref[0], (((1,), (1,)), ((), ())),
            preferred_element_type=jnp.float32)


def _final_body(o0a_ref, o1a_ref, o0b_ref, o1b_ref, w0_ref, w1_ref,
                wout_ref, out_ref):
    y = ((o0a_ref[...] + o0b_ref[...]) * w0_ref[:, 0:1]
         + (o1a_ref[...] + o1b_ref[...]) * w1_ref[:, 0:1])
    out_ref[...] = _dotT(y, wout_ref[...])


@jax.jit
def kernel(x, W_in, W_out, W_gate, gate_w, up_w, down_w):
    h, logits = pl.pallas_call(
        _router_body,
        grid=(T // BT,),
        in_specs=[
            pl.BlockSpec((BT, H), lambda t: (t, 0)),
            pl.BlockSpec((H, H), lambda t: (0, 0)),
            pl.BlockSpec((E, H), lambda t: (0, 0)),
        ],
        out_specs=[
            pl.BlockSpec((BT, H), lambda t: (t, 0)),
            pl.BlockSpec((BT, E), lambda t: (t, 0)),
        ],
        out_shape=[
            jax.ShapeDtypeStruct((T, H), jnp.float32),
            jax.ShapeDtypeStruct((T, E), jnp.float32),
        ],
    )(x, W_in, W_gate)

    pos0b, pos1b, w0b, w1b, te2, used2 = pl.pallas_call(
        _index_body,
        grid=(1,),
        in_specs=[pl.BlockSpec((T, E), lambda i: (0, 0))],
        out_specs=[
            pl.BlockSpec((T, E), lambda i: (0, 0)),
            pl.BlockSpec((T, E), lambda i: (0, 0)),
            pl.BlockSpec((T, E), lambda i: (0, 0)),
            pl.BlockSpec((T, E), lambda i: (0, 0)),
            pl.BlockSpec((1, G), lambda i: (0, 0)),
            pl.BlockSpec((1, E), lambda i: (0, 0)),
        ],
        out_shape=[
            jax.ShapeDtypeStruct((T, E), jnp.int32),
            jax.ShapeDtypeStruct((T, E), jnp.int32),
            jax.ShapeDtypeStruct((T, E), jnp.float32),
            jax.ShapeDtypeStruct((T, E), jnp.float32),
            jax.ShapeDtypeStruct((1, G), jnp.int32),
            jax.ShapeDtypeStruct((1, E), jnp.int32),
        ],
    )(logits)

    pos0 = pos0b[:, 0].reshape(1, T)
    pos1 = pos1b[:, 0].reshape(1, T)
    te = te2.reshape(G)
    used = used2[0, 0:1]

    hg = _dispatch(h, pos0, pos1)

    def _half_specs(fidx):
        return [
            pl.BlockSpec(
                (BROW, H),
                lambda g, te_r, u_r: (jnp.minimum(g, u_r[0] - 1), 0)),
            pl.BlockSpec(
                (1, BF, H), lambda g, te_r, u_r: (te_r[g], fidx, 0)),
            pl.BlockSpec(
                (1, BF, H), lambda g, te_r, u_r: (te_r[g], fidx, 0)),
            pl.BlockSpec(
                (1, H, BF), lambda g, te_r, u_r: (te_r[g], 0, fidx)),
        ]

    _og_spec = pl.BlockSpec(
        (BROW, H), lambda g, te_r, u_r: (jnp.where(g < u_r[0], g, G), 0))
    _og_shape = jax.ShapeDtypeStruct((NPAD + BROW, H), jnp.float32)

    og_a = pl.pallas_call(
        _grouped_body_first,
        grid_spec=pltpu.PrefetchScalarGridSpec(
            num_scalar_prefetch=2, grid=(G,),
            in_specs=_half_specs(0), out_specs=_og_spec),
        out_shape=_og_shape,
    )(te, used, hg, gate_w, up_w, down_w)

    og_b = pl.pallas_call(
        _grouped_body_first,
        grid_spec=pltpu.PrefetchScalarGridSpec(
            num_scalar_prefetch=2, grid=(G,),
            in_specs=_half_specs(1), out_specs=_og_spec),
        out_shape=_og_shape,
    )(te, used, hg, gate_w, up_w, down_w)

    o0a, o1a, o0b, o1b = _combine_gather(og_a, og_b, pos0, pos1)

    out = pl.pallas_call(
        _final_body,
        grid=(T // BT,),
        in_specs=[
            pl.BlockSpec((BT, H), lambda t: (t, 0)),
            pl.BlockSpec((BT, H), lambda t: (t, 0)),
            pl.BlockSpec((BT, H), lambda t: (t, 0)),
            pl.BlockSpec((BT, H), lambda t: (t, 0)),
            pl.BlockSpec((BT, E), lambda t: (t, 0)),
            pl.BlockSpec((BT, E), lambda t: (t, 0)),
            pl.BlockSpec((H, H), lambda t: (0, 0)),
        ],
        out_specs=pl.BlockSpec((BT, H), lambda t: (t, 0)),
        out_shape=jax.ShapeDtypeStruct((T, H), jnp.float32),
    )(o0a, o1a, o0b, o1b, w0b, w1b, W_out)
    return out


# speedup vs baseline: 1.0112x; 1.0112x over previous
import functools

import jax
import jax.numpy as jnp
from jax.experimental import pallas as pl
from jax.experimental.pallas import tpu as pltpu
from jax.experimental.pallas import tpu_sc as plsc

H = 1024
FFN = 4096
E = 8
K = 2
T = 4096

BT = 1024
BROW = 384
BF = 2048
G = -(-(T * K + E * (BROW - 1)) // BROW)
NPAD = G * BROW
NF = FFN // BF
CH = 128
NCH = T // CH
SCW = 32


def _dotT(a, b):
    return jax.lax.dot_general(a, b, (((1,), (1,)), ((), ())),
                               preferred_element_type=jnp.float32)


def _router_body(x_ref, win_ref, wgate_ref, h_ref, logits_ref):
    h = _dotT(x_ref[...], win_ref[...])
    h_ref[...] = h
    logits_ref[...] = _dotT(h, wgate_ref[...])


def _index_body(logits_ref, pos0_ref, pos1_ref, w0_ref, w1_ref, te_ref,
                used_ref):
    logits = logits_ref[...]
    probs = jax.nn.softmax(logits, axis=-1)
    lane = jax.lax.broadcasted_iota(jnp.int32, probs.shape, 1)
    p1 = jnp.max(probs, axis=-1, keepdims=True)
    i1 = jnp.argmax(probs, axis=-1)
    oh1 = (lane == i1[:, None]).astype(jnp.float32)
    masked = jnp.where(oh1 > 0, -1.0, probs)
    p2 = jnp.max(masked, axis=-1, keepdims=True)
    i2 = jnp.argmax(masked, axis=-1)
    oh2 = (lane == i2[:, None]).astype(jnp.float32)
    e2 = jnp.exp(p2 - p1)
    w1v = 1.0 / (1.0 + e2)
    w2v = e2 / (1.0 + e2)
    w0_ref[...] = jnp.broadcast_to(w1v, (T, E))
    w1_ref[...] = jnp.broadcast_to(w2v, (T, E))

    oh = oh1 + oh2
    sub = jax.lax.broadcasted_iota(jnp.int32, (CH, CH), 0)
    lan2 = jax.lax.broadcasted_iota(jnp.int32, (CH, CH), 1)
    tri_incl = (lan2 <= sub).astype(jnp.float32)
    incl_chunks = []
    totals = []
    for c in range(NCH):
        blk = oh[c * CH:(c + 1) * CH, :]
        incl = jnp.dot(tri_incl, blk, preferred_element_type=jnp.float32)
        incl_chunks.append(incl)
        totals.append(incl[CH - 1:CH, :])
    tot = jnp.concatenate(totals, axis=0)
    sub3 = jax.lax.broadcasted_iota(jnp.int32, (NCH, NCH), 0)
    lan3 = jax.lax.broadcasted_iota(jnp.int32, (NCH, NCH), 1)
    tri_strict = (lan3 < sub3).astype(jnp.float32)
    excl_tot = jnp.dot(tri_strict, tot, preferred_element_type=jnp.float32)
    incl_all = jnp.concatenate(
        [incl_chunks[c] + excl_tot[c:c + 1, :] for c in range(NCH)], axis=0)
    counts = jnp.sum(tot, axis=0, keepdims=True)
    padded = jnp.floor((counts + (BROW - 1)) / BROW) * BROW
    sub4 = jax.lax.broadcasted_iota(jnp.int32, (E, E), 0)
    lan4 = jax.lax.broadcasted_iota(jnp.int32, (E, E), 1)
    lt = (sub4 < lan4).astype(jnp.float32)
    offsets = jnp.dot(padded, lt, preferred_element_type=jnp.float32)
    excl = incl_all - oh
    pos_e = offsets + excl
    pos0 = jnp.sum(oh1 * pos_e, axis=-1, keepdims=True)
    pos1 = jnp.sum(oh2 * pos_e, axis=-1, keepdims=True)
    pos0_ref[...] = jnp.broadcast_to(pos0, (T, E)).astype(jnp.int32)
    pos1_ref[...] = jnp.broadcast_to(pos1, (T, E)).astype(jnp.int32)
    total_used = jnp.sum(padded, axis=-1, keepdims=True)
    used_ref[...] = jnp.broadcast_to(
        total_used / BROW, (1, E)).astype(jnp.int32)
    g_iota = (jax.lax.broadcasted_iota(jnp.int32, (1, G), 1) * BROW
              ).astype(jnp.float32)
    te = jnp.zeros((1, G), jnp.float32)
    for e in range(E):
        te = te + (g_iota >= offsets[0, e]).astype(jnp.float32)
    te_ref[...] = (te - 1.0).astype(jnp.int32)


NSHARD = 32
NTOK = T // NSHARD
NSUB = NTOK // SCW


def _dispatch(h, pos0, pos1):
    mesh = plsc.VectorSubcoreMesh(core_axis_name="c", subcore_axis_name="s")

    @functools.partial(
        pl.kernel,
        out_type=jax.ShapeDtypeStruct((NPAD, H), jnp.float32),
        mesh=mesh,
        scratch_types=[pltpu.VMEM((1, NTOK), jnp.int32),
                       pltpu.VMEM((1, NTOK), jnp.int32),
                       pltpu.VMEM((SCW, H), jnp.float32),
                       pltpu.VMEM((SCW, H), jnp.float32),
                       pltpu.SemaphoreType.DMA,
                       pltpu.SemaphoreType.DMA,
                       pltpu.SemaphoreType.DMA])
    def run(h_hbm, p0_hbm, p1_hbm, hg_hbm, i0b, i1b, rba, rbb, seml,
            sem0, sem1):
        c = jax.lax.axis_index("c")
        s = jax.lax.axis_index("s")
        base = (c * 16 + s) * NTOK
        pltpu.async_copy(p0_hbm.at[:, pl.ds(base, NTOK)], i0b, sem0).wait()
        pltpu.async_copy(p1_hbm.at[:, pl.ds(base, NTOK)], i1b, sem1).wait()

        bufs = [rba, rbb]
        ld = pltpu.async_copy(h_hbm.at[pl.ds(base, SCW), :], rba, seml)
        for sub in range(NSUB):
            cur = bufs[sub % 2]
            ld.wait()
            if sub + 1 < NSUB:
                ld = pltpu.async_copy(
                    h_hbm.at[pl.ds(base + (sub + 1) * SCW, SCW), :],
                    bufs[(sub + 1) % 2], seml)
            cp0 = pltpu.async_copy(
                cur, hg_hbm.at[i0b.at[0, pl.ds(sub * SCW, SCW)]], sem0)
            cp1 = pltpu.async_copy(
                cur, hg_hbm.at[i1b.at[0, pl.ds(sub * SCW, SCW)]], sem1)
            cp0.wait()
            cp1.wait()

    return run(h, pos0, pos1)


def _combine_gather(og_a, og_b, pos0, pos1):
    mesh = plsc.VectorSubcoreMesh(core_axis_name="c", subcore_axis_name="s")
    W4 = 8

    @functools.partial(
        pl.kernel,
        out_type=[jax.ShapeDtypeStruct((T, H), jnp.float32)] * 4,
        mesh=mesh,
        scratch_types=[pltpu.VMEM((1, NTOK), jnp.int32),
                       pltpu.VMEM((1, NTOK), jnp.int32)]
                      + [pltpu.VMEM((W4, H), jnp.float32)] * 8
                      + [pltpu.SemaphoreType.DMA] * 8)
    def run(oga_hbm, ogb_hbm, p0_hbm, p1_hbm, o0a_hbm, o1a_hbm, o0b_hbm,
            o1b_hbm, i0b, i1b, *bufsem):
        bufs = [bufsem[0:4], bufsem[4:8]]
        sems = [bufsem[8:12], bufsem[12:16]]
        outs = (o0a_hbm, o1a_hbm, o0b_hbm, o1b_hbm)
        c = jax.lax.axis_index("c")
        s = jax.lax.axis_index("s")
        base = (c * 16 + s) * NTOK
        pltpu.async_copy(p0_hbm.at[:, pl.ds(base, NTOK)], i0b,
                         bufsem[8]).wait()
        pltpu.async_copy(p1_hbm.at[:, pl.ds(base, NTOK)], i1b,
                         bufsem[9]).wait()

        nrounds = NTOK // W4

        def gathers(sub, bset, sset):
            i0 = i0b.at[0, pl.ds(sub * W4, W4)]
            i1 = i1b.at[0, pl.ds(sub * W4, W4)]
            srcs = (oga_hbm.at[i0], oga_hbm.at[i1],
                    ogb_hbm.at[i0], ogb_hbm.at[i1])
            return [pltpu.async_copy(src, buf, sem)
                    for src, buf, sem in zip(srcs, bset, sset)]

        gth = gathers(0, bufs[0], sems[0])
        sts = []
        for sub in range(nrounds):
            par = sub % 2
            for cp in gth:
                cp.wait()
            for cp in sts:
                cp.wait()
            if sub + 1 < nrounds:
                gth = gathers(sub + 1, bufs[1 - par], sems[1 - par])
            r0 = base + sub * W4
            sts = [pltpu.async_copy(buf, out.at[pl.ds(r0, W4), :], sem)
                   for buf, out, sem in zip(bufs[par], outs, sems[par])]
        for cp in sts:
            cp.wait()

    return run(og_a, og_b, pos0, pos1)


def _grouped_body_first(te_ref, used_ref, hg_ref, gate_ref, up_ref, down_ref,
                        og_ref):
    g = pl.program_id(0)

    @pl.when(g < used_ref[0])
    def _():
        hg = hg_ref[...]
        gv = _dotT(hg, gate_ref[0])
        uv = _dotT(hg, up_ref[0])
        p = jax.nn.silu(gv) * uv
        og_ref[...] = jax.lax.dot_general(
            p, down_ref[0], (((1,), (1,)), ((), ())),
            preferred_element_type=jnp.float32)


def _final_body(o0a_ref, o1a_ref, o0b_ref, o1b_ref, w0_ref, w1_ref,
                wout_ref, out_ref):
    y = ((o0a_ref[...] + o0b_ref[...]) * w0_ref[:, 0:1]
         + (o1a_ref[...] + o1b_ref[...]) * w1_ref[:, 0:1])
    out_ref[...] = _dotT(y, wout_ref[...])


@jax.jit
def kernel(x, W_in, W_out, W_gate, gate_w, up_w, down_w):
    h, logits = pl.pallas_call(
        _router_body,
        grid=(T // BT,),
        in_specs=[
            pl.BlockSpec((BT, H), lambda t: (t, 0)),
            pl.BlockSpec((H, H), lambda t: (0, 0)),
            pl.BlockSpec((E, H), lambda t: (0, 0)),
        ],
        out_specs=[
            pl.BlockSpec((BT, H), lambda t: (t, 0)),
            pl.BlockSpec((BT, E), lambda t: (t, 0)),
        ],
        out_shape=[
            jax.ShapeDtypeStruct((T, H), jnp.float32),
            jax.ShapeDtypeStruct((T, E), jnp.float32),
        ],
    )(x, W_in, W_gate)

    pos0b, pos1b, w0b, w1b, te2, used2 = pl.pallas_call(
        _index_body,
        grid=(1,),
        in_specs=[pl.BlockSpec((T, E), lambda i: (0, 0))],
        out_specs=[
            pl.BlockSpec((T, E), lambda i: (0, 0)),
            pl.BlockSpec((T, E), lambda i: (0, 0)),
            pl.BlockSpec((T, E), lambda i: (0, 0)),
            pl.BlockSpec((T, E), lambda i: (0, 0)),
            pl.BlockSpec((1, G), lambda i: (0, 0)),
            pl.BlockSpec((1, E), lambda i: (0, 0)),
        ],
        out_shape=[
            jax.ShapeDtypeStruct((T, E), jnp.int32),
            jax.ShapeDtypeStruct((T, E), jnp.int32),
            jax.ShapeDtypeStruct((T, E), jnp.float32),
            jax.ShapeDtypeStruct((T, E), jnp.float32),
            jax.ShapeDtypeStruct((1, G), jnp.int32),
            jax.ShapeDtypeStruct((1, E), jnp.int32),
        ],
    )(logits)

    pos0 = pos0b[:, 0].reshape(1, T)
    pos1 = pos1b[:, 0].reshape(1, T)
    te = te2.reshape(G)
    used = used2[0, 0:1]

    hg = _dispatch(h, pos0, pos1)

    def _half_specs(fidx):
        return [
            pl.BlockSpec(
                (BROW, H),
                lambda g, te_r, u_r: (jnp.minimum(g, u_r[0] - 1), 0)),
            pl.BlockSpec(
                (1, BF, H), lambda g, te_r, u_r: (te_r[g], fidx, 0)),
            pl.BlockSpec(
                (1, BF, H), lambda g, te_r, u_r: (te_r[g], fidx, 0)),
            pl.BlockSpec(
                (1, H, BF), lambda g, te_r, u_r: (te_r[g], 0, fidx)),
        ]

    _og_spec = pl.BlockSpec(
        (BROW, H), lambda g, te_r, u_r: (jnp.where(g < u_r[0], g, G), 0))
    _og_shape = jax.ShapeDtypeStruct((NPAD + BROW, H), jnp.float32)

    og_a = pl.pallas_call(
        _grouped_body_first,
        grid_spec=pltpu.PrefetchScalarGridSpec(
            num_scalar_prefetch=2, grid=(G,),
            in_specs=_half_specs(0), out_specs=_og_spec),
        out_shape=_og_shape,
    )(te, used, hg, gate_w, up_w, down_w)

    og_b = pl.pallas_call(
        _grouped_body_first,
        grid_spec=pltpu.PrefetchScalarGridSpec(
            num_scalar_prefetch=2, grid=(G,),
            in_specs=_half_specs(1), out_specs=_og_spec),
        out_shape=_og_shape,
    )(te, used, hg, gate_w, up_w, down_w)

    o0a, o1a, o0b, o1b = _combine_gather(og_a, og_b, pos0, pos1)

    out = pl.pallas_call(
        _final_body,
        grid=(T // BT,),
        in_specs=[
            pl.BlockSpec((BT, H), lambda t: (t, 0)),
            pl.BlockSpec((BT, H), lambda t: (t, 0)),
            pl.BlockSpec((BT, H), lambda t: (t, 0)),
            pl.BlockSpec((BT, H), lambda t: (t, 0)),
            pl.BlockSpec((BT, E), lambda t: (t, 0)),
            pl.BlockSpec((BT, E), lambda t: (t, 0)),
            pl.BlockSpec((H, H), lambda t: (0, 0)),
        ],
        out_specs=pl.BlockSpec((BT, H), lambda t: (t, 0)),
        out_shape=jax.ShapeDtypeStruct((T, H), jnp.float32),
    )(o0a, o1a, o0b, o1b, w0b, w1b, W_out)
    return out
